# Initial kernel scaffold; baseline (speedup 1.0000x reference)
import jax
import jax.numpy as jnp
from jax.experimental import pallas as pl


def kernel(x, edge_index, edge_type, conv1_w, conv2_w, lin1_w, lin1_b, lin2_w, lin2_b):
    raise NotImplementedError("write your pallas kernel here")


# speedup vs baseline: 5.1709x; 5.1709x over previous
import functools

import jax
import jax.numpy as jnp
from jax import lax
from jax.experimental import pallas as pl
from jax.experimental.pallas import tpu as pltpu
from jax.experimental.pallas import tpu_sc as plsc

N = 10000
E = 320000
C = 4
NPAD = 10240
NT = 32
RANGE = NPAD // NT
BLK = RANGE // 4
NBLK = NPAD // BLK
MAINCAP = 12288
SUBCAP = 4096
LSLACK = 64
CH = 2000
GB = 32
NEG = -3.0e38


def _popcount(m):
    return plsc.all_reduce_population_count(m)[0]


def _process_block(x_hbm, agg_hbm, ssrc, slidx, acc, rows, gsems, cs, gb, D):
    negv = jnp.full((16,), NEG, jnp.float32)

    def initacc(i, _):
        acc[pl.ds(i * 16, 16)] = negv
        return 0
    lax.fori_loop(0, RANGE * D // 16, initacc, 0)

    nb = (cs + GB - 1) // GB

    def _fire(k, b):
        pltpu.async_copy(x_hbm.at[ssrc.at[pl.ds(k * GB, GB)]], rows[b],
                         gsems[b])

    def _wait(b):
        pltpu.make_async_copy(
            x_hbm.at[ssrc.at[pl.ds(0, GB)]], rows[b], gsems[b]).wait()

    @pl.when(nb > 0)
    def _():
        _fire(0, 0)

    def gpair(p, _):
        for b in (0, 1):
            k = 2 * p + b

            @pl.when(k < nb)
            def _():
                _wait(b)

                @pl.when(k + 1 < nb)
                def _():
                    _fire(k + 1, b ^ 1)

                rb = rows[b]
                for g in range(GB // 16):
                    lv = slidx[pl.ds(k * GB + g * 16, 16)]
                    for e in range(16):
                        base = lv[e] * D
                        for j in range(D // 16):
                            sl = pl.ds(base + j * 16, 16)
                            acc[sl] = jnp.maximum(
                                acc[sl], rb[g * 16 + e, pl.ds(j * 16, 16)])
        return 0
    lax.fori_loop(0, (nb + 1) // 2, gpair, 0)

    def fix(i, _):
        v = acc[pl.ds(i * 16, 16)]
        acc[pl.ds(i * 16, 16)] = jnp.where(v < -1e37, 0.0, v)
        return 0
    lax.fori_loop(0, RANGE * D // 16, fix, 0)

    for t in range(C):
        pltpu.sync_copy(
            acc.at[pl.ds(t * BLK * D, BLK * D)],
            agg_hbm.at[pl.ds((t * NPAD + gb * BLK) * D, BLK * D)])


@functools.cache
def _build_conv1(D):
    mesh = plsc.VectorSubcoreMesh(
        core_axis_name="c", subcore_axis_name="s", num_cores=2,
        num_subcores=16)
    out_types = (
        jax.ShapeDtypeStruct((C * NPAD * D,), jnp.float32),
        jax.ShapeDtypeStruct((NBLK, SUBCAP), jnp.int32),
        jax.ShapeDtypeStruct((NBLK, SUBCAP), jnp.int32),
        jax.ShapeDtypeStruct((NT, 16), jnp.int32),
    )
    scratch = [
        pltpu.VMEM((2 * CH,), jnp.int32),
        pltpu.VMEM((2 * CH,), jnp.int32),
        pltpu.VMEM((2 * CH,), jnp.int32),
        pltpu.VMEM((MAINCAP + 16,), jnp.int32),
        pltpu.VMEM((MAINCAP + 16,), jnp.int32),
        pltpu.VMEM((SUBCAP + LSLACK,), jnp.int32),
        pltpu.VMEM((SUBCAP + LSLACK,), jnp.int32),
        pltpu.VMEM(((RANGE + 1) * D,), jnp.float32),
        pltpu.VMEM((GB, D), jnp.float32),
        pltpu.VMEM((GB, D), jnp.float32),
        pltpu.VMEM((16,), jnp.int32),
        pltpu.SemaphoreType.DMA,
        pltpu.SemaphoreType.DMA,
        pltpu.SemaphoreType.DMA,
        pltpu.SemaphoreType.DMA,
    ]

    @functools.partial(
        pl.kernel, out_type=out_types, mesh=mesh, scratch_types=scratch,
        compiler_params=pltpu.CompilerParams(needs_layout_passes=False))
    def conv1(x_hbm, src_hbm, dst_hbm, typ_hbm,
              agg_hbm, srcl_hbm, lidxl_hbm, cnts_hbm,
              src_st, dst_st, typ_st, msrc, mcode, ssrc, slidx,
              acc, rows0, rows1, cntv, st_sem0, st_sem1, g_sem0, g_sem1):
        tid = lax.axis_index("s") * 2 + lax.axis_index("c")
        tbase = tid * RANGE
        iota = lax.iota(jnp.int32, 16)

        def pre_s(i, _):
            ssrc[pl.ds(i * 16, 16)] = (iota + i * 16) & 8191
            return 0
        lax.fori_loop(0, (SUBCAP + LSLACK) // 16, pre_s, 0)
        sent = jnp.full((16,), 1 << 20, jnp.int32)

        def pre_m(i, _):
            mcode[pl.ds(i * 16, 16)] = sent
            return 0
        lax.fori_loop(0, (MAINCAP + 16) // 16, pre_m, 0)

        stage_sems = (st_sem0, st_sem1)

        def _fire_stage(ci, b):
            off = ci * CH
            pltpu.async_copy(src_hbm.at[pl.ds(off, CH)],
                             src_st.at[pl.ds(b * CH, CH)], stage_sems[b])
            pltpu.async_copy(dst_hbm.at[pl.ds(off, CH)],
                             dst_st.at[pl.ds(b * CH, CH)], stage_sems[b])
            pltpu.async_copy(typ_hbm.at[pl.ds(off, CH)],
                             typ_st.at[pl.ds(b * CH, CH)], stage_sems[b])

        def _wait_stage(b):
            for hbm, st in ((src_hbm, src_st), (dst_hbm, dst_st),
                            (typ_hbm, typ_st)):
                pltpu.make_async_copy(hbm.at[pl.ds(0, CH)],
                                      st.at[pl.ds(b * CH, CH)],
                                      stage_sems[b]).wait()

        NCH = E // CH
        _fire_stage(0, 0)

        def scan_pair(p, cnt):
            for b in (0, 1):
                ci = 2 * p + b
                _wait_stage(b)

                @pl.when(ci + 1 < NCH)
                def _():
                    _fire_stage(ci + 1, b ^ 1)

                def sub(i, cnt):
                    o = b * CH + i * 16
                    s = src_st[pl.ds(o, 16)]
                    d = dst_st[pl.ds(o, 16)]
                    t = typ_st[pl.ds(o, 16)]
                    rel = d - tbase
                    m = (rel >= 0) & (rel < RANGE)
                    code = rel * 4 + t
                    plsc.store_compressed(msrc.at[pl.ds(cnt, 16)], s, mask=m)
                    plsc.store_compressed(mcode.at[pl.ds(cnt, 16)], code,
                                          mask=m)
                    return jnp.minimum(cnt + _popcount(m), MAINCAP)
                cnt = lax.fori_loop(0, CH // 16, sub, cnt)
            return cnt
        cnt_main = lax.fori_loop(0, NCH // 2, scan_pair, jnp.int32(0))

        rows = (rows0, rows1)
        gsems = (g_sem0, g_sem1)
        dummy = jnp.full((16,), RANGE, jnp.int32)

        def do_block(blk, cntvec):
            gb = tid * 4 + blk
            lo = blk * 4 * BLK

            def subc(i, cs):
                s = msrc[pl.ds(i * 16, 16)]
                code = mcode[pl.ds(i * 16, 16)]
                cc = code - lo
                m = (cc >= 0) & (cc < 4 * BLK)
                lidx = (cc & 3) * BLK + (cc >> 2)
                plsc.store_compressed(ssrc.at[pl.ds(cs, 16)], s, mask=m)
                plsc.store_compressed(slidx.at[pl.ds(cs, 16)], lidx, mask=m)
                return jnp.minimum(cs + _popcount(m), SUBCAP)
            cs = lax.fori_loop(0, (cnt_main + 15) // 16, subc, jnp.int32(0))

            slidx[pl.ds(cs, 16)] = dummy
            slidx[pl.ds(cs + 16, 16)] = dummy
            pltpu.sync_copy(ssrc.at[pl.ds(0, SUBCAP)], srcl_hbm.at[gb])
            pltpu.sync_copy(slidx.at[pl.ds(0, SUBCAP)], lidxl_hbm.at[gb])

            _process_block(x_hbm, agg_hbm, ssrc, slidx, acc, rows, gsems,
                           cs, gb, D)
            return jnp.where(iota == blk, cs, cntvec)

        cntvec = lax.fori_loop(0, 4, do_block, jnp.zeros((16,), jnp.int32))
        cntv[...] = cntvec
        pltpu.sync_copy(cntv, cnts_hbm.at[tid])

    return conv1


@functools.cache
def _build_conv2(D):
    mesh = plsc.VectorSubcoreMesh(
        core_axis_name="c", subcore_axis_name="s", num_cores=2,
        num_subcores=16)
    out_types = jax.ShapeDtypeStruct((C * NPAD * D,), jnp.float32)
    scratch = [
        pltpu.VMEM((SUBCAP + LSLACK,), jnp.int32),
        pltpu.VMEM((SUBCAP + LSLACK,), jnp.int32),
        pltpu.VMEM(((RANGE + 1) * D,), jnp.float32),
        pltpu.VMEM((GB, D), jnp.float32),
        pltpu.VMEM((GB, D), jnp.float32),
        pltpu.VMEM((16,), jnp.int32),
        pltpu.SemaphoreType.DMA,
        pltpu.SemaphoreType.DMA,
    ]

    @functools.partial(
        pl.kernel, out_type=out_types, mesh=mesh, scratch_types=scratch,
        compiler_params=pltpu.CompilerParams(needs_layout_passes=False))
    def conv2(h_hbm, srcl_hbm, lidxl_hbm, cnts_hbm, agg_hbm,
              ssrc, slidx, acc, rows0, rows1, cntv, g_sem0, g_sem1):
        tid = lax.axis_index("s") * 2 + lax.axis_index("c")
        iota = lax.iota(jnp.int32, 16)
        pltpu.sync_copy(cnts_hbm.at[tid], cntv)
        cv = cntv[pl.ds(0, 16)]
        rows = (rows0, rows1)
        gsems = (g_sem0, g_sem1)

        def do_block(blk, _):
            gb = tid * 4 + blk
            pltpu.sync_copy(srcl_hbm.at[gb], ssrc.at[pl.ds(0, SUBCAP)])
            pltpu.sync_copy(lidxl_hbm.at[gb], slidx.at[pl.ds(0, SUBCAP)])
            cs = jnp.sum(jnp.where(iota == blk, cv, 0))
            _process_block(h_hbm, agg_hbm, ssrc, slidx, acc, rows, gsems,
                           cs, gb, D)
            return 0
        lax.fori_loop(0, 4, do_block, 0)

    return conv2


def _dense(x, agg, wt, cwt, b2d, din, dout, act):
    R = 512

    def body(x_ref, agg_ref, w_ref, cw_ref, b_ref, o_ref):
        a = jnp.dot(x_ref[...], w_ref[...],
                    preferred_element_type=jnp.float32)
        for c in range(C):
            a = a + jnp.dot(agg_ref[c], cw_ref[c],
                            preferred_element_type=jnp.float32)
        a = a + b_ref[0:1, :]
        if act == "relu":
            o_ref[...] = jnp.maximum(a, 0.0)
        else:
            o_ref[...] = jax.nn.sigmoid(a - 10.0)

    return pl.pallas_call(
        body,
        grid=(NPAD // R,),
        in_specs=[
            pl.BlockSpec((R, din), lambda i: (i, 0)),
            pl.BlockSpec((C, R, din), lambda i: (0, i, 0)),
            pl.BlockSpec((din, dout), lambda i: (0, 0)),
            pl.BlockSpec((C, din, dout), lambda i: (0, 0, 0)),
            pl.BlockSpec((8, dout), lambda i: (0, 0)),
        ],
        out_specs=pl.BlockSpec((R, dout), lambda i: (i, 0)),
        out_shape=jax.ShapeDtypeStruct((NPAD, dout), jnp.float32),
    )(x, agg, wt, cwt, b2d)


def kernel(x, edge_index, edge_type, conv1_w, conv2_w,
           lin1_w, lin1_b, lin2_w, lin2_b):
    src = edge_index[0].astype(jnp.int32)
    dst = edge_index[1].astype(jnp.int32)
    typ = edge_type.astype(jnp.int32)
    D1 = x.shape[1]
    D2 = lin1_w.shape[0]

    agg1f, srcl, lidxl, cnts = _build_conv1(D1)(x, src, dst, typ)
    agg1 = agg1f.reshape(C, NPAD, D1)

    x_pad = jnp.pad(x, ((0, NPAD - N), (0, 0)))
    b1 = jnp.broadcast_to(lin1_b[None, :], (8, D2))
    h1 = _dense(x_pad, agg1, lin1_w.T, jnp.transpose(conv1_w, (0, 2, 1)),
                b1, D1, D2, "relu")

    agg2f = _build_conv2(D2)(h1, srcl, lidxl, cnts)
    agg2 = agg2f.reshape(C, NPAD, D2)

    b2 = jnp.broadcast_to(lin2_b[None, :], (8, D1))
    out = _dense(h1, agg2, lin2_w.T, jnp.transpose(conv2_w, (0, 2, 1)),
                 b2, D2, D1, "sigmoid")
    return out[:N]


# speedup vs baseline: 7.5876x; 1.4674x over previous
import functools

import jax
import jax.numpy as jnp
from jax import lax
from jax.experimental import pallas as pl
from jax.experimental.pallas import tpu as pltpu
from jax.experimental.pallas import tpu_sc as plsc

N = 10000
E = 320000
C = 4
NPAD = 10240
NT = 32
RANGE = NPAD // NT
BLK = RANGE // 4
NBLK = NPAD // BLK
MAINCAP = 12288
SUBCAP = 4096
LSLACK = 64
CH = 2000
GB = 32
NEG = -3.0e38
SENT = 1 << 30


def _popcount(m):
    return plsc.all_reduce_population_count(m)[0]


def _process_block(x_hbm, agg_hbm, ssrc, slidx, acc, rows, gsems, cs, gb, D):
    HD = D // 2
    negv = plsc.bitcast(jnp.full((32,), NEG, jnp.bfloat16), jnp.int32)

    def initacc(i, _):
        acc[pl.ds(i * 16, 16)] = negv
        return 0
    lax.fori_loop(0, RANGE * HD // 16, initacc, 0)

    nb = (cs + GB - 1) // GB

    def _fire(k, b):
        pltpu.async_copy(x_hbm.at[ssrc.at[pl.ds(k * GB, GB)]], rows[b],
                         gsems[b])

    def _wait(b):
        pltpu.make_async_copy(
            x_hbm.at[ssrc.at[pl.ds(0, GB)]], rows[b], gsems[b]).wait()

    @pl.when(nb > 0)
    def _():
        _fire(0, 0)

    def gpair(p, _):
        for b in (0, 1):
            k = 2 * p + b

            @pl.when(k < nb)
            def _():
                _wait(b)

                @pl.when(k + 1 < nb)
                def _():
                    _fire(k + 1, b ^ 1)

                rb = rows[b]
                for g in range(GB // 16):
                    lv = slidx[pl.ds(k * GB + g * 16, 16)]
                    for e in range(16):
                        base = lv[e] * HD
                        for j in range(HD // 16):
                            sl = pl.ds(base + j * 16, 16)
                            av = plsc.bitcast(acc[sl], jnp.bfloat16)
                            rv = plsc.bitcast(
                                rb[g * 16 + e, pl.ds(j * 16, 16)],
                                jnp.bfloat16)
                            acc[sl] = plsc.bitcast(
                                jnp.maximum(av, rv), jnp.int32)
        return 0
    lax.fori_loop(0, (nb + 1) // 2, gpair, 0)

    zero = jnp.zeros((32,), jnp.bfloat16)
    thresh = jnp.full((32,), -1e37, jnp.bfloat16)

    def fix(i, _):
        v = plsc.bitcast(acc[pl.ds(i * 16, 16)], jnp.bfloat16)
        acc[pl.ds(i * 16, 16)] = plsc.bitcast(
            jnp.where(v < thresh, zero, v), jnp.int32)
        return 0
    lax.fori_loop(0, RANGE * HD // 16, fix, 0)

    for t in range(C):
        pltpu.sync_copy(
            acc.at[pl.ds(t * BLK * HD, BLK * HD)],
            agg_hbm.at[pl.ds((t * NPAD + gb * BLK) * HD, BLK * HD)])


@functools.cache
def _build_conv1(D):
    mesh = plsc.VectorSubcoreMesh(
        core_axis_name="c", subcore_axis_name="s", num_cores=2,
        num_subcores=16)
    out_types = (
        jax.ShapeDtypeStruct((C * NPAD * D // 2,), jnp.int32),
        jax.ShapeDtypeStruct((NBLK, SUBCAP), jnp.int32),
        jax.ShapeDtypeStruct((NBLK, SUBCAP), jnp.int32),
        jax.ShapeDtypeStruct((NT, 16), jnp.int32),
    )
    scratch = [
        pltpu.VMEM((2 * CH,), jnp.int32),
        pltpu.VMEM((MAINCAP + 16,), jnp.int32),
        pltpu.VMEM((SUBCAP + LSLACK,), jnp.int32),
        pltpu.VMEM((SUBCAP + LSLACK,), jnp.int32),
        pltpu.VMEM(((RANGE + 1) * D // 2,), jnp.int32),
        pltpu.VMEM((GB, D // 2), jnp.int32),
        pltpu.VMEM((GB, D // 2), jnp.int32),
        pltpu.VMEM((16,), jnp.int32),
        pltpu.SemaphoreType.DMA,
        pltpu.SemaphoreType.DMA,
        pltpu.SemaphoreType.DMA,
        pltpu.SemaphoreType.DMA,
    ]

    @functools.partial(
        pl.kernel, out_type=out_types, mesh=mesh, scratch_types=scratch,
        compiler_params=pltpu.CompilerParams(
            needs_layout_passes=False, use_tc_tiling_on_sc=False))
    def conv1(x_hbm, ew_hbm,
              agg_hbm, srcl_hbm, lidxl_hbm, cnts_hbm,
              ew_st, mlist, ssrc, slidx,
              acc, rows0, rows1, cntv, st_sem0, st_sem1, g_sem0, g_sem1):
        tid = lax.axis_index("s") * 2 + lax.axis_index("c")
        tbase4 = tid * (RANGE * 4)
        iota = lax.iota(jnp.int32, 16)

        def pre_s(i, _):
            ssrc[pl.ds(i * 16, 16)] = (iota + i * 16) & 8191
            return 0
        lax.fori_loop(0, (SUBCAP + LSLACK) // 16, pre_s, 0)
        sent = jnp.full((16,), SENT, jnp.int32)

        def pre_m(i, _):
            mlist[pl.ds(i * 16, 16)] = sent
            return 0
        lax.fori_loop(0, (MAINCAP + 16) // 16, pre_m, 0)

        stage_sems = (st_sem0, st_sem1)

        def _fire_stage(ci, b):
            pltpu.async_copy(ew_hbm.at[pl.ds(ci * CH, CH)],
                             ew_st.at[pl.ds(b * CH, CH)], stage_sems[b])

        def _wait_stage(b):
            pltpu.make_async_copy(ew_hbm.at[pl.ds(0, CH)],
                                  ew_st.at[pl.ds(b * CH, CH)],
                                  stage_sems[b]).wait()

        NCH = E // CH
        _fire_stage(0, 0)

        def scan_pair(p, cnt):
            for b in (0, 1):
                ci = 2 * p + b
                _wait_stage(b)

                @pl.when(ci + 1 < NCH)
                def _():
                    _fire_stage(ci + 1, b ^ 1)

                def sub(i, cnt):
                    w = ew_st[pl.ds(b * CH + i * 16, 16)]
                    rel = (w >> 14) - tbase4
                    m = (rel >= 0) & (rel < RANGE * 4)
                    plsc.store_compressed(mlist.at[pl.ds(cnt, 16)], w, mask=m)
                    return jnp.minimum(cnt + _popcount(m), MAINCAP)
                cnt = lax.fori_loop(0, CH // 16, sub, cnt)
            return cnt
        cnt_main = lax.fori_loop(0, NCH // 2, scan_pair, jnp.int32(0))

        rows = (rows0, rows1)
        gsems = (g_sem0, g_sem1)
        dummy = jnp.full((16,), RANGE, jnp.int32)

        def do_block(blk, cntvec):
            gb = tid * 4 + blk
            lo = tbase4 + blk * 4 * BLK

            def subc(i, cs):
                w = mlist[pl.ds(i * 16, 16)]
                cc = (w >> 14) - lo
                m = (cc >= 0) & (cc < 4 * BLK)
                lidx = (cc & 3) * BLK + (cc >> 2)
                s = w & 16383
                plsc.store_compressed(ssrc.at[pl.ds(cs, 16)], s, mask=m)
                plsc.store_compressed(slidx.at[pl.ds(cs, 16)], lidx, mask=m)
                return jnp.minimum(cs + _popcount(m), SUBCAP)
            cs = lax.fori_loop(0, (cnt_main + 15) // 16, subc, jnp.int32(0))

            slidx[pl.ds(cs, 16)] = dummy
            slidx[pl.ds(cs + 16, 16)] = dummy
            pltpu.sync_copy(ssrc.at[pl.ds(0, SUBCAP)], srcl_hbm.at[gb])
            pltpu.sync_copy(slidx.at[pl.ds(0, SUBCAP)], lidxl_hbm.at[gb])

            _process_block(x_hbm, agg_hbm, ssrc, slidx, acc, rows, gsems,
                           cs, gb, D)
            return jnp.where(iota == blk, cs, cntvec)

        cntvec = lax.fori_loop(0, 4, do_block, jnp.zeros((16,), jnp.int32))
        cntv[...] = cntvec
        pltpu.sync_copy(cntv, cnts_hbm.at[tid])

    return conv1


@functools.cache
def _build_conv2(D):
    mesh = plsc.VectorSubcoreMesh(
        core_axis_name="c", subcore_axis_name="s", num_cores=2,
        num_subcores=16)
    out_types = jax.ShapeDtypeStruct((C * NPAD * D // 2,), jnp.int32)
    scratch = [
        pltpu.VMEM((SUBCAP + LSLACK,), jnp.int32),
        pltpu.VMEM((SUBCAP + LSLACK,), jnp.int32),
        pltpu.VMEM(((RANGE + 1) * D // 2,), jnp.int32),
        pltpu.VMEM((GB, D // 2), jnp.int32),
        pltpu.VMEM((GB, D // 2), jnp.int32),
        pltpu.VMEM((16,), jnp.int32),
        pltpu.SemaphoreType.DMA,
        pltpu.SemaphoreType.DMA,
    ]

    @functools.partial(
        pl.kernel, out_type=out_types, mesh=mesh, scratch_types=scratch,
        compiler_params=pltpu.CompilerParams(
            needs_layout_passes=False, use_tc_tiling_on_sc=False))
    def conv2(h_hbm, srcl_hbm, lidxl_hbm, cnts_hbm, agg_hbm,
              ssrc, slidx, acc, rows0, rows1, cntv, g_sem0, g_sem1):
        tid = lax.axis_index("s") * 2 + lax.axis_index("c")
        iota = lax.iota(jnp.int32, 16)
        pltpu.sync_copy(cnts_hbm.at[tid], cntv)
        cv = cntv[pl.ds(0, 16)]
        rows = (rows0, rows1)
        gsems = (g_sem0, g_sem1)

        def do_block(blk, _):
            gb = tid * 4 + blk
            pltpu.sync_copy(srcl_hbm.at[gb], ssrc.at[pl.ds(0, SUBCAP)])
            pltpu.sync_copy(lidxl_hbm.at[gb], slidx.at[pl.ds(0, SUBCAP)])
            cs = jnp.sum(jnp.where(iota == blk, cv, 0))
            _process_block(h_hbm, agg_hbm, ssrc, slidx, acc, rows, gsems,
                           cs, gb, D)
            return 0
        lax.fori_loop(0, 4, do_block, 0)

    return conv2


def _dense1(x, agg, wt, cwt, b2d, din, dout):
    R = 512

    def body(x_ref, agg_ref, w_ref, cw_ref, b_ref, o_ref, ob_ref):
        a = jnp.dot(x_ref[...], w_ref[...],
                    preferred_element_type=jnp.float32)
        for c in range(C):
            a = a + jnp.dot(agg_ref[c][...].astype(jnp.float32), cw_ref[c],
                            preferred_element_type=jnp.float32)
        h = jnp.maximum(a + b_ref[0:1, :], 0.0)
        o_ref[...] = h
        ob_ref[...] = h.astype(jnp.bfloat16)

    return pl.pallas_call(
        body,
        grid=(NPAD // R,),
        in_specs=[
            pl.BlockSpec((R, din), lambda i: (i, 0)),
            pl.BlockSpec((C, R, din), lambda i: (0, i, 0)),
            pl.BlockSpec((din, dout), lambda i: (0, 0)),
            pl.BlockSpec((C, din, dout), lambda i: (0, 0, 0)),
            pl.BlockSpec((8, dout), lambda i: (0, 0)),
        ],
        out_specs=[pl.BlockSpec((R, dout), lambda i: (i, 0)),
                   pl.BlockSpec((R, dout), lambda i: (i, 0))],
        out_shape=[jax.ShapeDtypeStruct((NPAD, dout), jnp.float32),
                   jax.ShapeDtypeStruct((NPAD, dout), jnp.bfloat16)],
    )(x, agg, wt, cwt, b2d)


def _dense2(x, agg, wt, cwt, b2d, din, dout):
    R = 512

    def body(x_ref, agg_ref, w_ref, cw_ref, b_ref, o_ref):
        a = jnp.dot(x_ref[...], w_ref[...],
                    preferred_element_type=jnp.float32)
        for c in range(C):
            a = a + jnp.dot(agg_ref[c][...].astype(jnp.float32), cw_ref[c],
                            preferred_element_type=jnp.float32)
        o_ref[...] = jax.nn.sigmoid(a + b_ref[0:1, :] - 10.0)

    return pl.pallas_call(
        body,
        grid=(NPAD // R,),
        in_specs=[
            pl.BlockSpec((R, din), lambda i: (i, 0)),
            pl.BlockSpec((C, R, din), lambda i: (0, i, 0)),
            pl.BlockSpec((din, dout), lambda i: (0, 0)),
            pl.BlockSpec((C, din, dout), lambda i: (0, 0, 0)),
            pl.BlockSpec((8, dout), lambda i: (0, 0)),
        ],
        out_specs=pl.BlockSpec((R, dout), lambda i: (i, 0)),
        out_shape=jax.ShapeDtypeStruct((NPAD, dout), jnp.float32),
    )(x, agg, wt, cwt, b2d)


def kernel(x, edge_index, edge_type, conv1_w, conv2_w,
           lin1_w, lin1_b, lin2_w, lin2_b):
    src = edge_index[0].astype(jnp.int32)
    dst = edge_index[1].astype(jnp.int32)
    typ = edge_type.astype(jnp.int32)
    ew = src | ((dst * 4 + typ) << 14)
    D1 = x.shape[1]
    D2 = lin1_w.shape[0]
    x_g = lax.bitcast_convert_type(
        x.astype(jnp.bfloat16).reshape(N, D1 // 2, 2), jnp.int32)

    agg1f, srcl, lidxl, cnts = _build_conv1(D1)(x_g, ew)
    agg1 = lax.bitcast_convert_type(
        agg1f.reshape(C, NPAD, D1 // 2), jnp.bfloat16).reshape(C, NPAD, D1)

    x_pad = jnp.pad(x, ((0, NPAD - N), (0, 0)))
    b1 = jnp.broadcast_to(lin1_b[None, :], (8, D2))
    h1, h1_bf = _dense1(x_pad, agg1, lin1_w.T,
                        jnp.transpose(conv1_w, (0, 2, 1)), b1, D1, D2)

    h1_g = lax.bitcast_convert_type(
        h1_bf.reshape(NPAD, D2 // 2, 2), jnp.int32)
    agg2f = _build_conv2(D2)(h1_g, srcl, lidxl, cnts)
    agg2 = lax.bitcast_convert_type(
        agg2f.reshape(C, NPAD, D2 // 2), jnp.bfloat16).reshape(C, NPAD, D2)

    b2 = jnp.broadcast_to(lin2_b[None, :], (8, D1))
    out = _dense2(h1, agg2, lin2_w.T, jnp.transpose(conv2_w, (0, 2, 1)),
                  b2, D2, D1)
    return out[:N]


# speedup vs baseline: 8.3858x; 1.1052x over previous
import functools

import jax
import jax.numpy as jnp
from jax import lax
from jax.experimental import pallas as pl
from jax.experimental.pallas import tpu as pltpu
from jax.experimental.pallas import tpu_sc as plsc

N = 10000
E = 320000
C = 4
NPAD = 10240
NT = 32
RANGE = NPAD // NT
BLK = RANGE // 4
NBLK = NPAD // BLK
MAINCAP = 12288
SUBCAP = 4096
LSLACK = 64
CH = 2000
GB = 32
HD = 128
NEG = -3.0e38
SENT = 1 << 30


def _popcount(m):
    return plsc.all_reduce_population_count(m)[0]


def _process_block(x_hbm, agg_hbm, ssrc, slidx, acc, rows, gsems, cs, gb,
                   packed):
    if packed:
        negv = plsc.bitcast(jnp.full((32,), NEG, jnp.bfloat16), jnp.int32)
    else:
        negv = jnp.full((16,), NEG, jnp.float32)

    def initacc(i, _):
        for j in range(HD // 16):
            acc[i, pl.ds(j * 16, 16)] = negv
        return 0
    lax.fori_loop(0, RANGE, initacc, 0)

    nb = (cs + GB - 1) // GB

    def _fire(k, b):
        pltpu.async_copy(x_hbm.at[ssrc.at[pl.ds(k * GB, GB)]], rows[b],
                         gsems[b])

    def _wait(b):
        pltpu.make_async_copy(
            x_hbm.at[ssrc.at[pl.ds(0, GB)]], rows[b], gsems[b]).wait()

    @pl.when(nb > 0)
    def _():
        _fire(0, 0)

    def gpair(p, _):
        for b in (0, 1):
            k = 2 * p + b

            @pl.when(k < nb)
            def _():
                _wait(b)

                @pl.when(k + 1 < nb)
                def _():
                    _fire(k + 1, b ^ 1)

                rb = rows[b]
                for g in range(GB // 16):
                    lv = slidx[pl.ds(k * GB + g * 16, 16)]
                    for e in range(16):
                        li = lv[e]
                        for j in range(HD // 16):
                            sl = pl.ds(j * 16, 16)
                            av = acc[li, sl]
                            rv = rb[g * 16 + e, sl]
                            if packed:
                                mx = plsc.bitcast(jnp.maximum(
                                    plsc.bitcast(av, jnp.bfloat16),
                                    plsc.bitcast(rv, jnp.bfloat16)),
                                    jnp.int32)
                            else:
                                mx = jnp.maximum(av, rv)
                            acc[li, sl] = mx
        return 0
    lax.fori_loop(0, (nb + 1) // 2, gpair, 0)

    if packed:
        zero = jnp.zeros((32,), jnp.bfloat16)
        thresh = jnp.full((32,), -1e37, jnp.bfloat16)
    else:
        zero = jnp.zeros((16,), jnp.float32)
        thresh = jnp.full((16,), -1e37, jnp.float32)

    def fix(i, _):
        for j in range(HD // 16):
            sl = pl.ds(j * 16, 16)
            v = acc[i, sl]
            if packed:
                vb = plsc.bitcast(v, jnp.bfloat16)
                acc[i, sl] = plsc.bitcast(
                    jnp.where(vb < thresh, zero, vb), jnp.int32)
            else:
                acc[i, sl] = jnp.where(v < thresh, zero, v)
        return 0
    lax.fori_loop(0, RANGE, fix, 0)

    for t in range(C):
        pltpu.sync_copy(
            acc.at[pl.ds(t * BLK, BLK)],
            agg_hbm.at[pl.ds(t * NPAD + gb * BLK, BLK)])


@functools.cache
def _build_conv1():
    mesh = plsc.VectorSubcoreMesh(
        core_axis_name="c", subcore_axis_name="s", num_cores=2,
        num_subcores=16)
    out_types = (
        jax.ShapeDtypeStruct((C * NPAD, HD), jnp.float32),
        jax.ShapeDtypeStruct((NBLK, SUBCAP), jnp.int32),
        jax.ShapeDtypeStruct((NBLK, SUBCAP), jnp.int32),
        jax.ShapeDtypeStruct((NT, 16), jnp.int32),
    )
    scratch = [
        pltpu.VMEM((2 * CH,), jnp.int32),
        pltpu.VMEM((MAINCAP + 16,), jnp.int32),
        pltpu.VMEM((SUBCAP + LSLACK,), jnp.int32),
        pltpu.VMEM((SUBCAP + LSLACK,), jnp.int32),
        pltpu.VMEM((RANGE + 1, HD), jnp.float32),
        pltpu.VMEM((GB, HD), jnp.float32),
        pltpu.VMEM((GB, HD), jnp.float32),
        pltpu.VMEM((16,), jnp.int32),
        pltpu.SemaphoreType.DMA,
        pltpu.SemaphoreType.DMA,
        pltpu.SemaphoreType.DMA,
        pltpu.SemaphoreType.DMA,
    ]

    @functools.partial(
        pl.kernel, out_type=out_types, mesh=mesh, scratch_types=scratch,
        compiler_params=pltpu.CompilerParams(needs_layout_passes=False))
    def conv1(x_hbm, ew_hbm,
              agg_hbm, srcl_hbm, lidxl_hbm, cnts_hbm,
              ew_st, mlist, ssrc, slidx,
              acc, rows0, rows1, cntv, st_sem0, st_sem1, g_sem0, g_sem1):
        tid = lax.axis_index("s") * 2 + lax.axis_index("c")
        tbase4 = tid * (RANGE * 4)
        iota = lax.iota(jnp.int32, 16)

        def pre_s(i, _):
            ssrc[pl.ds(i * 16, 16)] = (iota + i * 16) & 8191
            return 0
        lax.fori_loop(0, (SUBCAP + LSLACK) // 16, pre_s, 0)
        sent = jnp.full((16,), SENT, jnp.int32)

        def pre_m(i, _):
            mlist[pl.ds(i * 16, 16)] = sent
            return 0
        lax.fori_loop(0, (MAINCAP + 16) // 16, pre_m, 0)

        stage_sems = (st_sem0, st_sem1)

        def _fire_stage(ci, b):
            pltpu.async_copy(ew_hbm.at[pl.ds(ci * CH, CH)],
                             ew_st.at[pl.ds(b * CH, CH)], stage_sems[b])

        def _wait_stage(b):
            pltpu.make_async_copy(ew_hbm.at[pl.ds(0, CH)],
                                  ew_st.at[pl.ds(b * CH, CH)],
                                  stage_sems[b]).wait()

        NCH = E // CH
        _fire_stage(0, 0)

        def scan_pair(p, cnt):
            for b in (0, 1):
                ci = 2 * p + b
                _wait_stage(b)

                @pl.when(ci + 1 < NCH)
                def _():
                    _fire_stage(ci + 1, b ^ 1)

                def sub(i, cnt):
                    w = ew_st[pl.ds(b * CH + i * 16, 16)]
                    rel = (w >> 14) - tbase4
                    m = (rel >= 0) & (rel < RANGE * 4)
                    plsc.store_compressed(mlist.at[pl.ds(cnt, 16)], w, mask=m)
                    return jnp.minimum(cnt + _popcount(m), MAINCAP)
                cnt = lax.fori_loop(0, CH // 16, sub, cnt)
            return cnt
        cnt_main = lax.fori_loop(0, NCH // 2, scan_pair, jnp.int32(0))

        rows = (rows0, rows1)
        gsems = (g_sem0, g_sem1)
        dummy = jnp.full((16,), RANGE, jnp.int32)

        def do_block(blk, cntvec):
            gb = tid * 4 + blk
            lo = tbase4 + blk * 4 * BLK

            def subc(i, cs):
                w = mlist[pl.ds(i * 16, 16)]
                cc = (w >> 14) - lo
                m = (cc >= 0) & (cc < 4 * BLK)
                lidx = (cc & 3) * BLK + (cc >> 2)
                s = w & 16383
                plsc.store_compressed(ssrc.at[pl.ds(cs, 16)], s, mask=m)
                plsc.store_compressed(slidx.at[pl.ds(cs, 16)], lidx, mask=m)
                return jnp.minimum(cs + _popcount(m), SUBCAP)
            cs = lax.fori_loop(0, (cnt_main + 15) // 16, subc, jnp.int32(0))

            slidx[pl.ds(cs, 16)] = dummy
            slidx[pl.ds(cs + 16, 16)] = dummy
            pltpu.sync_copy(ssrc.at[pl.ds(0, SUBCAP)], srcl_hbm.at[gb])
            pltpu.sync_copy(slidx.at[pl.ds(0, SUBCAP)], lidxl_hbm.at[gb])

            _process_block(x_hbm, agg_hbm, ssrc, slidx, acc, rows, gsems,
                           cs, gb, packed=False)
            return jnp.where(iota == blk, cs, cntvec)

        cntvec = lax.fori_loop(0, 4, do_block, jnp.zeros((16,), jnp.int32))
        cntv[...] = cntvec
        pltpu.sync_copy(cntv, cnts_hbm.at[tid])

    return conv1


@functools.cache
def _build_conv2():
    mesh = plsc.VectorSubcoreMesh(
        core_axis_name="c", subcore_axis_name="s", num_cores=2,
        num_subcores=16)
    out_types = jax.ShapeDtypeStruct((C * NPAD, HD), jnp.int32)
    scratch = [
        pltpu.VMEM((SUBCAP + LSLACK,), jnp.int32),
        pltpu.VMEM((SUBCAP + LSLACK,), jnp.int32),
        pltpu.VMEM((RANGE + 1, HD), jnp.int32),
        pltpu.VMEM((GB, HD), jnp.int32),
        pltpu.VMEM((GB, HD), jnp.int32),
        pltpu.VMEM((16,), jnp.int32),
        pltpu.SemaphoreType.DMA,
        pltpu.SemaphoreType.DMA,
    ]

    @functools.partial(
        pl.kernel, out_type=out_types, mesh=mesh, scratch_types=scratch,
        compiler_params=pltpu.CompilerParams(needs_layout_passes=False))
    def conv2(h_hbm, srcl_hbm, lidxl_hbm, cnts_hbm, agg_hbm,
              ssrc, slidx, acc, rows0, rows1, cntv, g_sem0, g_sem1):
        tid = lax.axis_index("s") * 2 + lax.axis_index("c")
        iota = lax.iota(jnp.int32, 16)
        pltpu.sync_copy(cnts_hbm.at[tid], cntv)
        cv = cntv[pl.ds(0, 16)]
        rows = (rows0, rows1)
        gsems = (g_sem0, g_sem1)

        def do_block(blk, _):
            gb = tid * 4 + blk
            pltpu.sync_copy(srcl_hbm.at[gb], ssrc.at[pl.ds(0, SUBCAP)])
            pltpu.sync_copy(lidxl_hbm.at[gb], slidx.at[pl.ds(0, SUBCAP)])
            cs = jnp.sum(jnp.where(iota == blk, cv, 0))
            _process_block(h_hbm, agg_hbm, ssrc, slidx, acc, rows, gsems,
                           cs, gb, packed=True)
            return 0
        lax.fori_loop(0, 4, do_block, 0)

    return conv2


def _dense1(x, agg, wt, cwt, b2d):
    R = 512

    def body(x_ref, agg_ref, w_ref, cw_ref, b_ref, o_ref, op_ref):
        a = jnp.dot(x_ref[...], w_ref[...],
                    preferred_element_type=jnp.float32)
        for c in range(C):
            a = a + jnp.dot(agg_ref[c], cw_ref[c],
                            preferred_element_type=jnp.float32)
        h = jnp.maximum(a + b_ref[0:1, :], 0.0)
        o_ref[...] = h
        u0 = lax.bitcast_convert_type(h[:, :HD], jnp.int32)
        u1 = lax.bitcast_convert_type(h[:, HD:], jnp.int32)
        r0 = (u0 + 0x7FFF + ((u0 >> 16) & 1)) >> 16
        r1 = (u1 + 0x7FFF + ((u1 >> 16) & 1)) >> 16
        op_ref[...] = (r0 & 0xFFFF) | (r1 << 16)

    return pl.pallas_call(
        body,
        grid=(NPAD // R,),
        in_specs=[
            pl.BlockSpec((R, 128), lambda i: (i, 0)),
            pl.BlockSpec((C, R, 128), lambda i: (0, i, 0)),
            pl.BlockSpec((128, 256), lambda i: (0, 0)),
            pl.BlockSpec((C, 128, 256), lambda i: (0, 0, 0)),
            pl.BlockSpec((8, 256), lambda i: (0, 0)),
        ],
        out_specs=[pl.BlockSpec((R, 256), lambda i: (i, 0)),
                   pl.BlockSpec((R, HD), lambda i: (i, 0))],
        out_shape=[jax.ShapeDtypeStruct((NPAD, 256), jnp.float32),
                   jax.ShapeDtypeStruct((NPAD, HD), jnp.int32)],
    )(x, agg, wt, cwt, b2d)


def _dense2(x, agg, wt, cwt, b2d):
    R = 512

    def body(x_ref, agg_ref, w_ref, cw_ref, b_ref, o_ref):
        a = jnp.dot(x_ref[...], w_ref[...],
                    preferred_element_type=jnp.float32)
        for c in range(C):
            w = agg_ref[c]
            low = lax.bitcast_convert_type(w << 16, jnp.float32)
            high = lax.bitcast_convert_type(w & ~0xFFFF, jnp.float32)
            a = a + jnp.dot(low, cw_ref[c][:HD],
                            preferred_element_type=jnp.float32)
            a = a + jnp.dot(high, cw_ref[c][HD:],
                            preferred_element_type=jnp.float32)
        o_ref[...] = jax.nn.sigmoid(a + b_ref[0:1, :] - 10.0)

    return pl.pallas_call(
        body,
        grid=(NPAD // R,),
        in_specs=[
            pl.BlockSpec((R, 256), lambda i: (i, 0)),
            pl.BlockSpec((C, R, HD), lambda i: (0, i, 0)),
            pl.BlockSpec((256, 128), lambda i: (0, 0)),
            pl.BlockSpec((C, 256, 128), lambda i: (0, 0, 0)),
            pl.BlockSpec((8, 128), lambda i: (0, 0)),
        ],
        out_specs=pl.BlockSpec((R, 128), lambda i: (i, 0)),
        out_shape=jax.ShapeDtypeStruct((NPAD, 128), jnp.float32),
    )(x, agg, wt, cwt, b2d)


def kernel(x, edge_index, edge_type, conv1_w, conv2_w,
           lin1_w, lin1_b, lin2_w, lin2_b):
    src = edge_index[0].astype(jnp.int32)
    dst = edge_index[1].astype(jnp.int32)
    typ = edge_type.astype(jnp.int32)
    ew = src | ((dst * 4 + typ) << 14)

    agg1f, srcl, lidxl, cnts = _build_conv1()(x, ew)
    agg1 = agg1f.reshape(C, NPAD, 128)

    x_pad = jnp.pad(x, ((0, NPAD - N), (0, 0)))
    b1 = jnp.broadcast_to(lin1_b[None, :], (8, 256))
    h1, h1p = _dense1(x_pad, agg1, lin1_w.T,
                      jnp.transpose(conv1_w, (0, 2, 1)), b1)

    agg2f = _build_conv2()(h1p, srcl, lidxl, cnts)
    agg2 = agg2f.reshape(C, NPAD, HD)

    b2 = jnp.broadcast_to(lin2_b[None, :], (8, 128))
    out = _dense2(h1, agg2, lin2_w.T, jnp.transpose(conv2_w, (0, 2, 1)), b2)
    return out[:N]
